# baseline (speedup 1.0000x reference)
import functools

import jax
import jax.numpy as jnp
from jax import lax
from jax.experimental import pallas as pl
from jax.experimental.pallas import tpu as pltpu
from jax.experimental.pallas import tpu_sc as plsc

TOTAL = 32768
D = 1024
NSEG = 16
L = 16
NC = 2
NS = 16
NW = NC * NS
CPW = D // NW

RB = 2048
RBS = 64
NB = TOTAL // RBS
SUB = RB // RBS


def _tc_block_reduce(values):
    def body(v_ref, bm_ref, ba_ref):
        base = pl.program_id(0) * RB
        for sb in range(SUB):
            y = v_ref[pl.ds(sb * RBS, RBS), :]
            m = jnp.max(y, axis=0, keepdims=True)
            rows = lax.broadcasted_iota(jnp.int32, (RBS, D), 0)
            cand = jnp.where(y == m, rows, TOTAL)
            a = jnp.min(cand, axis=0, keepdims=True)
            bm_ref[sb] = m
            ba_ref[sb] = a + (base + sb * RBS)

    return pl.pallas_call(
        body,
        grid=(TOTAL // RB,),
        in_specs=[pl.BlockSpec((RB, D), lambda i: (i, 0))],
        out_specs=[pl.BlockSpec((SUB, 1, D), lambda i: (i, 0, 0)),
                   pl.BlockSpec((SUB, 1, D), lambda i: (i, 0, 0))],
        out_shape=[jax.ShapeDtypeStruct((NB, 1, D), jnp.float32),
                   jax.ShapeDtypeStruct((NB, 1, D), jnp.int32)],
    )(values)


def _sc_merge(values, ps_pad, bm, ba):
    mesh = plsc.VectorSubcoreMesh(core_axis_name="c", subcore_axis_name="s")

    @functools.partial(
        pl.kernel,
        mesh=mesh,
        out_type=jax.ShapeDtypeStruct((NSEG, D), jnp.int32),
        scratch_types=[
            pltpu.VMEM((NB, CPW), jnp.float32),
            pltpu.VMEM((NB, CPW), jnp.int32),
            pltpu.VMEM((NSEG, RBS, CPW), jnp.float32),
            pltpu.VMEM((NSEG, RBS, CPW), jnp.float32),
            pltpu.VMEM((NSEG, CPW), jnp.int32),
            pltpu.VMEM((32,), jnp.int32),
            pltpu.SemaphoreType.DMA,
        ],
        compiler_params=pltpu.CompilerParams(use_tc_tiling_on_sc=False),
    )
    def body(values_hbm, ps_hbm, bm_hbm, ba_hbm, out_hbm,
             mbuf, abuf, hbuf, tbuf, outv, ps_v, sem):
        wid = lax.axis_index("s") * NC + lax.axis_index("c")
        c0 = wid * CPW
        pltpu.sync_copy(ps_hbm, ps_v)
        psa = ps_v[pl.ds(0, L)]
        psb = ps_v[pl.ds(L, L)]
        bounds = [psa[i] for i in range(L)] + [psb[0]]

        fb, lb, head_end, tail_lo, hst, tst = [], [], [], [], [], []
        for s in range(NSEG):
            lo, hi = bounds[s], bounds[s + 1]
            f = lax.div(lo + (RBS - 1), RBS)
            b = lax.div(hi, RBS)
            fb.append(f)
            lb.append(b)
            head_end.append(jnp.minimum(f * RBS, hi))
            t = jnp.where(b > f, b * RBS, hi)
            tail_lo.append(t)
            hst.append(jnp.minimum(lo, TOTAL - RBS))
            tst.append(jnp.minimum(t, TOTAL - RBS))

        pltpu.async_copy(bm_hbm.at[:, pl.ds(c0, CPW)], mbuf, sem)
        pltpu.async_copy(ba_hbm.at[:, pl.ds(c0, CPW)], abuf, sem)
        for s in range(NSEG):
            pltpu.async_copy(
                values_hbm.at[pl.ds(hst[s], RBS), pl.ds(c0, CPW)],
                hbuf.at[s], sem)
            pltpu.async_copy(
                values_hbm.at[pl.ds(tst[s], RBS), pl.ds(c0, CPW)],
                tbuf.at[s], sem)
        pltpu.make_async_copy(bm_hbm.at[:, pl.ds(c0, CPW)], mbuf, sem).wait()
        pltpu.make_async_copy(ba_hbm.at[:, pl.ds(c0, CPW)], abuf, sem).wait()
        for s in range(NSEG):
            pltpu.make_async_copy(
                values_hbm.at[pl.ds(0, RBS), pl.ds(c0, CPW)],
                hbuf.at[s], sem).wait()
            pltpu.make_async_copy(
                values_hbm.at[pl.ds(0, RBS), pl.ds(c0, CPW)],
                tbuf.at[s], sem).wait()

        for s in range(NSEG):
            lo, hi = bounds[s], bounds[s + 1]

            def raw_scan(buf_s, st, j_lo, j_hi, carry):
                def rbody(j, carry):
                    m0, m1, i0, i1 = carry
                    v0 = buf_s[j, pl.ds(0, L)]
                    v1 = buf_s[j, pl.ds(L, L)]
                    r = jnp.full((L,), st + j, jnp.int32)
                    g0 = v0 > m0
                    g1 = v1 > m1
                    return (jnp.where(g0, v0, m0), jnp.where(g1, v1, m1),
                            jnp.where(g0, r, i0), jnp.where(g1, r, i1))
                return lax.fori_loop(j_lo, j_hi, rbody, carry)

            carry = (jnp.full((L,), -jnp.inf, jnp.float32),
                     jnp.full((L,), -jnp.inf, jnp.float32),
                     jnp.full((L,), lo, jnp.int32),
                     jnp.full((L,), lo, jnp.int32))
            carry = raw_scan(hbuf.at[s], hst[s], lo - hst[s],
                             head_end[s] - hst[s], carry)

            def bbody(r, carry):
                m0, m1, i0, i1 = carry
                v0 = mbuf[r, pl.ds(0, L)]
                v1 = mbuf[r, pl.ds(L, L)]
                a0 = abuf[r, pl.ds(0, L)]
                a1 = abuf[r, pl.ds(L, L)]
                g0 = v0 > m0
                g1 = v1 > m1
                return (jnp.where(g0, v0, m0), jnp.where(g1, v1, m1),
                        jnp.where(g0, a0, i0), jnp.where(g1, a1, i1))

            carry = lax.fori_loop(fb[s], jnp.maximum(lb[s], fb[s]),
                                  bbody, carry)

            m0, m1, i0, i1 = raw_scan(tbuf.at[s], tst[s], tail_lo[s] - tst[s],
                                      hi - tst[s], carry)
            outv[s, pl.ds(0, L)] = i0
            outv[s, pl.ds(L, L)] = i1

        pltpu.sync_copy(outv, out_hbm.at[:, pl.ds(c0, CPW)])

    return body(values, ps_pad, bm, ba)


def kernel(values, prefix_sum):
    bm, ba = _tc_block_reduce(values)
    ps_pad = jnp.zeros((32,), jnp.int32).at[: NSEG + 1].set(prefix_sum)
    return _sc_merge(values, ps_pad, bm.reshape(NB, D), ba.reshape(NB, D))


# speedup vs baseline: 1.0330x; 1.0330x over previous
import functools

import jax
import jax.numpy as jnp
from jax import lax
from jax.experimental import pallas as pl
from jax.experimental.pallas import tpu as pltpu
from jax.experimental.pallas import tpu_sc as plsc

TOTAL = 32768
D = 1024
NSEG = 16
L = 16
NC = 2
NS = 16
NW = NC * NS
CPW = D // NW

RB = 2048
RBS = 64
NB = TOTAL // RBS
SUB = RB // RBS
NSTEP = TOTAL // RB


def _tc_interior(values, ps):
    def body(ps_ref, v_ref, sm_ref, si_ref, pm, pa, stm, sti):
        i = pl.program_id(0)
        base = i * RB

        @pl.when(i == 0)
        def _():
            stm[...] = jnp.full((NSEG, D), -jnp.inf, jnp.float32)
            sti[...] = jnp.zeros((NSEG, D), jnp.int32)

        for sb in range(SUB):
            y = v_ref[pl.ds(sb * RBS, RBS), :]
            m = jnp.max(y, axis=0, keepdims=True)
            rows = lax.broadcasted_iota(jnp.int32, (RBS, D), 0)
            cand = jnp.where(y == m, rows, TOTAL)
            a = jnp.min(cand, axis=0, keepdims=True)
            pm[pl.ds(sb, 1), :] = m
            pa[pl.ds(sb, 1), :] = a + (base + sb * RBS)

        sbi = lax.broadcasted_iota(jnp.int32, (SUB, D), 0)
        for s in range(NSEG):
            lo = ps_ref[s]
            hi = ps_ref[s + 1]
            a_s = jnp.maximum(lax.div(lo + (RBS - 1), RBS) * RBS, base)
            b_s = jnp.minimum(lax.div(hi, RBS) * RBS, base + RB)

            @pl.when(b_s > a_s)
            def _(s=s, a_s=a_s, b_s=b_s):
                sb_lo = lax.div(a_s - base, RBS)
                sb_hi = lax.div(b_s - base, RBS)
                msk = (sbi >= sb_lo) & (sbi < sb_hi)
                x = pm[...]
                m = jnp.max(jnp.where(msk, x, -jnp.inf), axis=0,
                            keepdims=True)
                a = jnp.min(jnp.where(msk & (x == m), pa[...], TOTAL),
                            axis=0, keepdims=True)
                g = m > stm[pl.ds(s, 1), :]
                stm[pl.ds(s, 1), :] = jnp.where(g, m, stm[pl.ds(s, 1), :])
                sti[pl.ds(s, 1), :] = jnp.where(g, a, sti[pl.ds(s, 1), :])

        @pl.when(i == NSTEP - 1)
        def _():
            sm_ref[...] = stm[...]
            si_ref[...] = sti[...]

    return pl.pallas_call(
        body,
        grid_spec=pltpu.PrefetchScalarGridSpec(
            num_scalar_prefetch=1,
            grid=(NSTEP,),
            in_specs=[pl.BlockSpec((RB, D), lambda i, ps: (i, 0))],
            out_specs=[pl.BlockSpec((NSEG, D), lambda i, ps: (0, 0)),
                       pl.BlockSpec((NSEG, D), lambda i, ps: (0, 0))],
            scratch_shapes=[
                pltpu.VMEM((SUB, D), jnp.float32),
                pltpu.VMEM((SUB, D), jnp.int32),
                pltpu.VMEM((NSEG, D), jnp.float32),
                pltpu.VMEM((NSEG, D), jnp.int32),
            ],
        ),
        out_shape=[jax.ShapeDtypeStruct((NSEG, D), jnp.float32),
                   jax.ShapeDtypeStruct((NSEG, D), jnp.int32)],
    )(ps, values)


def _sc_merge(values, ps_pad, sm, si):
    mesh = plsc.VectorSubcoreMesh(core_axis_name="c", subcore_axis_name="s")

    @functools.partial(
        pl.kernel,
        mesh=mesh,
        out_type=jax.ShapeDtypeStruct((NSEG, D), jnp.int32),
        scratch_types=[
            pltpu.VMEM((NSEG, CPW), jnp.float32),
            pltpu.VMEM((NSEG, CPW), jnp.int32),
            pltpu.VMEM((NSEG, RBS, CPW), jnp.float32),
            pltpu.VMEM((NSEG, RBS, CPW), jnp.float32),
            pltpu.VMEM((NSEG, CPW), jnp.int32),
            pltpu.VMEM((32,), jnp.int32),
            pltpu.SemaphoreType.DMA,
        ],
        compiler_params=pltpu.CompilerParams(use_tc_tiling_on_sc=False),
    )
    def body(values_hbm, ps_hbm, sm_hbm, si_hbm, out_hbm,
             mbuf, abuf, hbuf, tbuf, outv, ps_v, sem):
        wid = lax.axis_index("s") * NC + lax.axis_index("c")
        c0 = wid * CPW
        pltpu.sync_copy(ps_hbm, ps_v)
        psa = ps_v[pl.ds(0, L)]
        psb = ps_v[pl.ds(L, L)]
        bounds = [psa[i] for i in range(L)] + [psb[0]]

        fb, lb, head_end, tail_lo, hst, tst = [], [], [], [], [], []
        for s in range(NSEG):
            lo, hi = bounds[s], bounds[s + 1]
            f = lax.div(lo + (RBS - 1), RBS)
            b = lax.div(hi, RBS)
            fb.append(f)
            lb.append(b)
            head_end.append(jnp.minimum(f * RBS, hi))
            t = jnp.where(b > f, b * RBS, hi)
            tail_lo.append(t)
            hst.append(jnp.minimum(lo, TOTAL - RBS))
            tst.append(jnp.minimum(t, TOTAL - RBS))

        pltpu.async_copy(sm_hbm.at[:, pl.ds(c0, CPW)], mbuf, sem)
        pltpu.async_copy(si_hbm.at[:, pl.ds(c0, CPW)], abuf, sem)
        for s in range(NSEG):
            pltpu.async_copy(
                values_hbm.at[pl.ds(hst[s], RBS), pl.ds(c0, CPW)],
                hbuf.at[s], sem)
            pltpu.async_copy(
                values_hbm.at[pl.ds(tst[s], RBS), pl.ds(c0, CPW)],
                tbuf.at[s], sem)
        pltpu.make_async_copy(sm_hbm.at[:, pl.ds(c0, CPW)], mbuf, sem).wait()
        pltpu.make_async_copy(si_hbm.at[:, pl.ds(c0, CPW)], abuf, sem).wait()
        for s in range(NSEG):
            pltpu.make_async_copy(
                values_hbm.at[pl.ds(0, RBS), pl.ds(c0, CPW)],
                hbuf.at[s], sem).wait()
            pltpu.make_async_copy(
                values_hbm.at[pl.ds(0, RBS), pl.ds(c0, CPW)],
                tbuf.at[s], sem).wait()

        for s in range(NSEG):
            lo, hi = bounds[s], bounds[s + 1]

            def raw_scan(buf_s, st, j_lo, j_hi, carry):
                def rbody(j, carry):
                    m0, m1, i0, i1 = carry
                    v0 = buf_s[j, pl.ds(0, L)]
                    v1 = buf_s[j, pl.ds(L, L)]
                    r = jnp.full((L,), st + j, jnp.int32)
                    g0 = v0 > m0
                    g1 = v1 > m1
                    return (jnp.where(g0, v0, m0), jnp.where(g1, v1, m1),
                            jnp.where(g0, r, i0), jnp.where(g1, r, i1))
                return lax.fori_loop(j_lo, j_hi, rbody, carry)

            carry = (jnp.full((L,), -jnp.inf, jnp.float32),
                     jnp.full((L,), -jnp.inf, jnp.float32),
                     jnp.full((L,), lo, jnp.int32),
                     jnp.full((L,), lo, jnp.int32))
            m0, m1, i0, i1 = raw_scan(hbuf.at[s], hst[s], lo - hst[s],
                                      head_end[s] - hst[s], carry)

            v0 = mbuf[s, pl.ds(0, L)]
            v1 = mbuf[s, pl.ds(L, L)]
            a0 = abuf[s, pl.ds(0, L)]
            a1 = abuf[s, pl.ds(L, L)]
            g0 = v0 > m0
            g1 = v1 > m1
            carry = (jnp.where(g0, v0, m0), jnp.where(g1, v1, m1),
                     jnp.where(g0, a0, i0), jnp.where(g1, a1, i1))

            m0, m1, i0, i1 = raw_scan(tbuf.at[s], tst[s], tail_lo[s] - tst[s],
                                      hi - tst[s], carry)
            outv[s, pl.ds(0, L)] = i0
            outv[s, pl.ds(L, L)] = i1

        pltpu.sync_copy(outv, out_hbm.at[:, pl.ds(c0, CPW)])

    return body(values, ps_pad, sm, si)


def kernel(values, prefix_sum):
    sm, si = _tc_interior(values, prefix_sum)
    ps_pad = jnp.zeros((32,), jnp.int32).at[: NSEG + 1].set(prefix_sum)
    return _sc_merge(values, ps_pad, sm, si)


# speedup vs baseline: 2.1932x; 2.1232x over previous
import functools

import jax
import jax.numpy as jnp
from jax import lax
from jax.experimental import pallas as pl
from jax.experimental.pallas import tpu as pltpu
from jax.experimental.pallas import tpu_sc as plsc

TOTAL = 32768
D = 1024
NSEG = 16
L = 16
NC = 2
NS = 16
SPS = 4
CG = 128
NCG = CG // L

RB = 2048
RBS = 64
SUB = RB // RBS
NSTEP = TOTAL // RB
HB = 72


def _tc_interior(values, ps):
    def body(ps_ref, v_ref, sm_ref, si_ref, pm, pa, stm, sti):
        i = pl.program_id(0)
        base = i * RB

        @pl.when(i == 0)
        def _():
            stm[...] = jnp.full((NSEG, D), -jnp.inf, jnp.float32)
            sti[...] = jnp.zeros((NSEG, D), jnp.int32)

        for sb in range(SUB):
            y = v_ref[pl.ds(sb * RBS, RBS), :]
            m = jnp.max(y, axis=0, keepdims=True)
            rows = lax.broadcasted_iota(jnp.int32, (RBS, D), 0)
            cand = jnp.where(y == m, rows, TOTAL)
            a = jnp.min(cand, axis=0, keepdims=True)
            pm[pl.ds(sb, 1), :] = m
            pa[pl.ds(sb, 1), :] = a + (base + sb * RBS)

        sbi = lax.broadcasted_iota(jnp.int32, (SUB, D), 0)
        for s in range(NSEG):
            lo = ps_ref[s]
            hi = ps_ref[s + 1]
            a_s = jnp.maximum(lax.div(lo + (RBS - 1), RBS) * RBS, base)
            b_s = jnp.minimum(lax.div(hi, RBS) * RBS, base + RB)

            @pl.when(b_s > a_s)
            def _(s=s, a_s=a_s, b_s=b_s):
                sb_lo = lax.div(a_s - base, RBS)
                sb_hi = lax.div(b_s - base, RBS)
                msk = (sbi >= sb_lo) & (sbi < sb_hi)
                x = pm[...]
                m = jnp.max(jnp.where(msk, x, -jnp.inf), axis=0,
                            keepdims=True)
                a = jnp.min(jnp.where(msk & (x == m), pa[...], TOTAL),
                            axis=0, keepdims=True)
                g = m > stm[pl.ds(s, 1), :]
                stm[pl.ds(s, 1), :] = jnp.where(g, m, stm[pl.ds(s, 1), :])
                sti[pl.ds(s, 1), :] = jnp.where(g, a, sti[pl.ds(s, 1), :])

        @pl.when(i == NSTEP - 1)
        def _():
            sm_ref[...] = stm[...]
            si_ref[...] = sti[...]

    return pl.pallas_call(
        body,
        grid_spec=pltpu.PrefetchScalarGridSpec(
            num_scalar_prefetch=1,
            grid=(NSTEP,),
            in_specs=[pl.BlockSpec((RB, D), lambda i, ps: (i, 0))],
            out_specs=[pl.BlockSpec((NSEG, D), lambda i, ps: (0, 0)),
                       pl.BlockSpec((NSEG, D), lambda i, ps: (0, 0))],
            scratch_shapes=[
                pltpu.VMEM((SUB, D), jnp.float32),
                pltpu.VMEM((SUB, D), jnp.int32),
                pltpu.VMEM((NSEG, D), jnp.float32),
                pltpu.VMEM((NSEG, D), jnp.int32),
            ],
        ),
        out_shape=[jax.ShapeDtypeStruct((NSEG, D), jnp.float32),
                   jax.ShapeDtypeStruct((NSEG, D), jnp.int32)],
    )(ps, values)


def _sc_merge(values, ps_pad, sm, si):
    mesh = plsc.VectorSubcoreMesh(core_axis_name="c", subcore_axis_name="s")

    @functools.partial(
        pl.kernel,
        mesh=mesh,
        out_type=jax.ShapeDtypeStruct((NSEG, D), jnp.int32),
        scratch_types=[
            pltpu.VMEM((NSEG, CG), jnp.float32),
            pltpu.VMEM((NSEG, CG), jnp.int32),
            pltpu.VMEM((SPS, HB, CG), jnp.float32),
            pltpu.VMEM((SPS, HB, CG), jnp.float32),
            pltpu.VMEM((SPS, CG), jnp.int32),
            pltpu.VMEM((32,), jnp.int32),
            pltpu.VMEM_SHARED((NSEG, NS * 32), jnp.int32),
            pltpu.SemaphoreType.DMA,
        ],
        compiler_params=pltpu.CompilerParams(needs_layout_passes=False),
    )
    def body(values_hbm, ps_hbm, sm_hbm, si_hbm, out_hbm,
             mbuf, abuf, hbuf, tbuf, outv, ps_v, shalf, sem):
        cid = lax.axis_index("c")
        u = lax.axis_index("s")
        gloc = lax.rem(u, SPS)
        sset = lax.div(u, SPS)
        c0 = pl.multiple_of((cid * SPS + gloc) * CG, CG)

        pltpu.sync_copy(ps_hbm, ps_v)
        sidx = sset + SPS * lax.iota(jnp.int32, L)
        lov = plsc.load_gather(ps_v, [jnp.minimum(sidx, NSEG)])
        hiv = plsc.load_gather(ps_v, [jnp.minimum(sidx + 1, NSEG)])
        los = [lov[k] for k in range(SPS)]
        his = [hiv[k] for k in range(SPS)]

        head_end, tail_lo, hst, tst = [], [], [], []
        for k in range(SPS):
            lo, hi = los[k], his[k]
            f = lax.div(lo + (RBS - 1), RBS)
            b = lax.div(hi, RBS)
            head_end.append(jnp.minimum(f * RBS, hi))
            t = jnp.where(b > f, b * RBS, hi)
            tail_lo.append(t)
            hs = jnp.minimum(lax.div(lo, 8) * 8, TOTAL - HB)
            ts = jnp.minimum(lax.div(t, 8) * 8, TOTAL - HB)
            hst.append(pl.multiple_of(hs, 8))
            tst.append(pl.multiple_of(ts, 8))

        pltpu.async_copy(sm_hbm.at[:, pl.ds(c0, CG)], mbuf, sem)
        pltpu.async_copy(si_hbm.at[:, pl.ds(c0, CG)], abuf, sem)
        for k in range(SPS):
            pltpu.async_copy(
                values_hbm.at[pl.ds(hst[k], HB), pl.ds(c0, CG)],
                hbuf.at[k], sem)
            pltpu.async_copy(
                values_hbm.at[pl.ds(tst[k], HB), pl.ds(c0, CG)],
                tbuf.at[k], sem)
        pltpu.make_async_copy(sm_hbm.at[:, pl.ds(c0, CG)], mbuf, sem).wait()
        pltpu.make_async_copy(si_hbm.at[:, pl.ds(c0, CG)], abuf, sem).wait()
        for k in range(SPS):
            pltpu.make_async_copy(
                values_hbm.at[pl.ds(0, HB), pl.ds(c0, CG)],
                hbuf.at[k], sem).wait()
            pltpu.make_async_copy(
                values_hbm.at[pl.ds(0, HB), pl.ds(c0, CG)],
                tbuf.at[k], sem).wait()

        for k in range(SPS):
            lo, hi = los[k], his[k]
            s = sset + SPS * k

            def raw_scan(buf_k, st, j_lo, j_hi, carry):
                def rbody(j, carry):
                    ms, idx = carry
                    r = jnp.full((L,), st + j, jnp.int32)
                    nms, nidx = [], []
                    for cg in range(NCG):
                        v = buf_k[j, pl.ds(cg * L, L)]
                        g = v > ms[cg]
                        nms.append(jnp.where(g, v, ms[cg]))
                        nidx.append(jnp.where(g, r, idx[cg]))
                    return tuple(nms), tuple(nidx)
                return lax.fori_loop(j_lo, j_hi, rbody, carry)

            carry = (tuple(jnp.full((L,), -jnp.inf, jnp.float32)
                           for _ in range(NCG)),
                     tuple(jnp.full((L,), lo, jnp.int32)
                           for _ in range(NCG)))
            carry = raw_scan(hbuf.at[k], hst[k], lo - hst[k],
                             head_end[k] - hst[k], carry)

            ms, idx = carry
            nms, nidx = [], []
            for cg in range(NCG):
                v = mbuf[s, pl.ds(cg * L, L)]
                a = abuf[s, pl.ds(cg * L, L)]
                g = v > ms[cg]
                nms.append(jnp.where(g, v, ms[cg]))
                nidx.append(jnp.where(g, a, idx[cg]))
            carry = (tuple(nms), tuple(nidx))

            ms, idx = raw_scan(tbuf.at[k], tst[k], tail_lo[k] - tst[k],
                               hi - tst[k], carry)
            for cg in range(NCG):
                outv[k, pl.ds(cg * L, L)] = idx[cg]

        for k in range(SPS):
            s = sset + SPS * k
            pltpu.sync_copy(outv.at[k],
                            shalf.at[s, pl.ds(gloc * CG, CG)])
        plsc.subcore_barrier()

        @pl.when(u == 0)
        def _():
            pltpu.sync_copy(
                shalf,
                out_hbm.at[:, pl.ds(pl.multiple_of(cid * (NS * 32), 512),
                                    NS * 32)])

    return body(values, ps_pad, sm, si)


def kernel(values, prefix_sum):
    sm, si = _tc_interior(values, prefix_sum)
    ps_pad = jnp.zeros((32,), jnp.int32).at[: NSEG + 1].set(prefix_sum)
    return _sc_merge(values, ps_pad, sm, si)
